# baseline (speedup 1.0000x reference)
import functools
import jax
import jax.numpy as jnp
from jax import lax
from jax.experimental import pallas as pl
from jax.experimental.pallas import tpu as pltpu
from jax.experimental.pallas import tpu_sc as plsc

BATCH = 1024
D = 1024
H = 675
HP = 768
NNUM = 32
NCAT = 8
CARD = 1000
NEXP = 8
MID = 512
RH = 256
EPS = 1e-5
_SQRT1_2 = 0.7071067811865476

NC, NS = 2, 16
NW = NC * NS
SPW = BATCH // NW
ROWS_PW = SPW * NCAT
CH_ROWS = 32
NCHUNK = ROWS_PW // CH_ROWS
SPC = CH_ROWS // NCAT
LGRP = HP // 16


def _gelu(x):
    return 0.5 * x * (1.0 + lax.erf(x * _SQRT1_2))


def _table_kernel(emb_ref, bias_ref, lng_ref, lnb_ref, w0vt_ref, b0v_ref,
                  sgugb_ref, scal_ref, out_ref):
    j = pl.program_id(0) // 5
    x = emb_ref[...] + bias_ref[0]
    m = jnp.mean(x, axis=-1, keepdims=True)
    v = jnp.mean((x - m) * (x - m), axis=-1, keepdims=True)
    y = (x - m) * lax.rsqrt(v + EPS) * lng_ref[...] + lnb_ref[...]
    z = jnp.dot(y, w0vt_ref[...], preferred_element_type=jnp.float32) + b0v_ref[...]
    h = _gelu(z)
    mh = jnp.sum(h, axis=-1, keepdims=True) * (1.0 / H)
    hc = h - mh
    vh = (jnp.sum(hc * hc, axis=-1, keepdims=True)
          - (HP - H) * mh * mh) * (1.0 / H)
    yv = hc * lax.rsqrt(vh + EPS) * sgugb_ref[0:1, :] + sgugb_ref[1:2, :]
    out_ref[...] = yv * scal_ref[j]


def _build_table(cat_emb, bias8, lng, lnb, w0vt, b0v, sgugb, scal):
    grid = 40
    rb = 200
    return pl.pallas_call(
        _table_kernel,
        grid=(grid,),
        in_specs=[
            pl.BlockSpec((rb, D), lambda i: (i, 0)),
            pl.BlockSpec((1, 1, D), lambda i: (i // 5, 0, 0)),
            pl.BlockSpec((1, D), lambda i: (0, 0)),
            pl.BlockSpec((1, D), lambda i: (0, 0)),
            pl.BlockSpec((D, HP), lambda i: (0, 0)),
            pl.BlockSpec((1, HP), lambda i: (0, 0)),
            pl.BlockSpec((2, HP), lambda i: (0, 0)),
            pl.BlockSpec(memory_space=pltpu.SMEM),
        ],
        out_specs=pl.BlockSpec((rb, HP), lambda i: (i, 0)),
        out_shape=jax.ShapeDtypeStruct((NCAT * CARD, HP), jnp.float32),
    )(cat_emb, bias8[:, None, :], lng, lnb, w0vt, b0v, sgugb, scal)


def _make_gather():
    mesh = plsc.VectorSubcoreMesh(core_axis_name="c", subcore_axis_name="s")

    @functools.partial(
        pl.kernel, mesh=mesh,
        out_type=jax.ShapeDtypeStruct((BATCH, HP), jnp.float32),
        scratch_types=[
            pltpu.VMEM((ROWS_PW,), jnp.int32),
            pltpu.VMEM((CH_ROWS, HP), jnp.float32),
            pltpu.VMEM((CH_ROWS, HP), jnp.float32),
            pltpu.VMEM((SPW, HP), jnp.float32),
            pltpu.SemaphoreType.DMA,
            pltpu.SemaphoreType.DMA,
        ],
    )
    def gather(table_hbm, fidx_hbm, out_hbm, idx_v, bufa, bufb, out_v, sema, semb):
        wid = lax.axis_index("s") * NC + lax.axis_index("c")
        base = wid * SPW
        pltpu.sync_copy(fidx_hbm.at[pl.ds(base * NCAT, ROWS_PW)], idx_v)
        bufs = (bufa, bufb)
        sems = (sema, semb)
        pltpu.make_async_copy(
            table_hbm.at[idx_v.at[pl.ds(0, CH_ROWS)]], bufs[0], sems[0]).start()
        for c in range(NCHUNK):
            if c + 1 < NCHUNK:
                pltpu.make_async_copy(
                    table_hbm.at[idx_v.at[pl.ds((c + 1) * CH_ROWS, CH_ROWS)]],
                    bufs[(c + 1) % 2], sems[(c + 1) % 2]).start()
            buf = bufs[c % 2]
            pltpu.make_async_copy(
                table_hbm.at[idx_v.at[pl.ds(c * CH_ROWS, CH_ROWS)]],
                buf, sems[c % 2]).wait()
            for s in range(SPC):
                r0 = s * NCAT
                orow = c * SPC + s

                def grp(gi, _):
                    sl = pl.ds(gi * 16, 16)
                    acc0 = buf[r0 + 0, sl] + buf[r0 + 1, sl]
                    acc1 = buf[r0 + 2, sl] + buf[r0 + 3, sl]
                    acc2 = buf[r0 + 4, sl] + buf[r0 + 5, sl]
                    acc3 = buf[r0 + 6, sl] + buf[r0 + 7, sl]
                    out_v[orow, sl] = (acc0 + acc1) + (acc2 + acc3)
                    return 0

                lax.fori_loop(0, LGRP, grp, 0)
        pltpu.sync_copy(out_v, out_hbm.at[pl.ds(base, SPW)])

    return gather


def _gather_fn(table, fidx):
    return _make_gather()(table, fidx)


def _numeric_kernel(xnum_ref, a1_ref, a2_ref, a34_ref, wg_ref, mom_ref,
                    constv_ref, out_ref):
    acc = jnp.broadcast_to(constv_ref[...], out_ref.shape)
    s_all = xnum_ref[...]
    a3 = a34_ref[0:1, :]
    a4 = a34_ref[1:2, :]
    for t in range(NNUM):
        s = s_all[:, t:t + 1]
        m = s * mom_ref[0, t] + mom_ref[1, t]
        var = s * s * mom_ref[2, t] + 2.0 * s * mom_ref[3, t] + mom_ref[4, t]
        rsinv = lax.rsqrt(var + EPS)
        z = (s * a1_ref[t:t + 1, :] + a2_ref[t:t + 1, :] - m * a3) * rsinv + a4
        h = _gelu(z)
        mh = jnp.sum(h, axis=-1, keepdims=True) * (1.0 / H)
        hc = h - mh
        vh = (jnp.sum(hc * hc, axis=-1, keepdims=True)
              - (HP - H) * mh * mh) * (1.0 / H)
        acc = acc + hc * lax.rsqrt(vh + EPS) * wg_ref[t:t + 1, :]
    out_ref[...] = acc


def _run_numeric(xnum, a1, a2, a34, wg, mom, constv):
    bb = 256
    c = lambda i: (0, 0)
    return pl.pallas_call(
        _numeric_kernel,
        grid=(BATCH // bb,),
        in_specs=[
            pl.BlockSpec((bb, NNUM), lambda i: (i, 0)),
            pl.BlockSpec((NNUM, HP), c),
            pl.BlockSpec((NNUM, HP), c),
            pl.BlockSpec((2, HP), c),
            pl.BlockSpec((NNUM, HP), c),
            pl.BlockSpec(memory_space=pltpu.SMEM),
            pl.BlockSpec((1, HP), c),
        ],
        out_specs=pl.BlockSpec((bb, HP), lambda i: (i, 0)),
        out_shape=jax.ShapeDtypeStruct((BATCH, HP), jnp.float32),
    )(xnum, a1, a2, a34, wg, mom, constv)


def _heads_kernel(numv_ref, catc_ref, w1ut_ref, big_ref, gw1t_ref, gvec_ref,
                  rw1t_ref, rvec_ref, rw2t_ref, aw1t_ref, sc3_ref, rbv_ref,
                  eb2s_ref, ew1t_ref, eb1_ref, ew2_ref, out_ref,
                  hid_s, rl_s, eo_s, bga_s):
    e = pl.program_id(1)

    @pl.when(e == 0)
    def _first():
        acc = catc_ref[...] + numv_ref[...]
        x0 = jnp.dot(acc, w1ut_ref[...], preferred_element_type=jnp.float32,
                     precision=lax.Precision.HIGHEST)
        x0 = x0 + big_ref[0:1, :]
        m0 = jnp.mean(x0, axis=-1, keepdims=True)
        v0 = jnp.mean((x0 - m0) * (x0 - m0), axis=-1, keepdims=True)
        hid = (x0 - m0) * lax.rsqrt(v0 + EPS) * big_ref[1:2, :] + big_ref[2:3, :]
        hid_s[...] = hid
        base = jnp.sum(hid * big_ref[3:4, :], axis=-1, keepdims=True) + sc3_ref[0]
        gr = jnp.maximum(
            jnp.dot(hid, gw1t_ref[...], preferred_element_type=jnp.float32)
            + gvec_ref[0:1, :], 0.0)
        gout = jnp.sum(gr * gvec_ref[1:2, :], axis=-1, keepdims=True) + sc3_ref[1]
        rr = jnp.maximum(
            jnp.dot(hid, rw1t_ref[...], preferred_element_type=jnp.float32)
            + rvec_ref[0:1, :], 0.0)
        rl = jnp.dot(rr, rw2t_ref[...], preferred_element_type=jnp.float32)
        rl_s[...] = rl + rbv_ref[...]
        ar = jnp.maximum(
            jnp.dot(hid, aw1t_ref[...], preferred_element_type=jnp.float32)
            + rvec_ref[1:2, :], 0.0)
        av = jnp.sum(ar * rvec_ref[2:3, :], axis=-1, keepdims=True) + sc3_ref[2]
        alpha = 1.0 / (1.0 + jnp.exp(-av))
        bga_s[...] = jnp.concatenate(
            [base + gout, alpha, jnp.zeros_like(base), jnp.zeros_like(base),
             base, base, base, base], axis=1)
        eo_s[...] = jnp.zeros(eo_s.shape, eo_s.dtype)

    eh = jnp.maximum(
        jnp.dot(hid_s[...], ew1t_ref[0], preferred_element_type=jnp.float32)
        + eb1_ref[0], 0.0)
    eoe = jnp.sum(eh * ew2_ref[0], axis=-1, keepdims=True) + eb2s_ref[e]
    onehot = (lax.broadcasted_iota(jnp.int32, eo_s.shape, 1) == e
              ).astype(jnp.float32)
    eo_s[...] += eoe * onehot

    @pl.when(e == NEXP - 1)
    def _last():
        rl = rl_s[...]
        eo = eo_s[...]
        tri = (lax.broadcasted_iota(jnp.int32, (NEXP, NEXP), 0)
               <= lax.broadcasted_iota(jnp.int32, (NEXP, NEXP), 1)
               ).astype(jnp.float32)
        m1 = jnp.max(rl, axis=-1, keepdims=True)
        eq1 = (rl == m1).astype(jnp.float32)
        f1 = eq1 * (jnp.dot(eq1, tri, preferred_element_type=jnp.float32)
                    == 1.0).astype(jnp.float32)
        rl2 = jnp.where(f1 > 0.0, -jnp.inf, rl)
        m2 = jnp.max(rl2, axis=-1, keepdims=True)
        eq2 = (rl2 == m2).astype(jnp.float32)
        f2 = eq2 * (jnp.dot(eq2, tri, preferred_element_type=jnp.float32)
                    == 1.0).astype(jnp.float32)
        sel1 = jnp.sum(eo * f1, axis=-1, keepdims=True)
        sel2 = jnp.sum(eo * f2, axis=-1, keepdims=True)
        e2 = jnp.exp(m2 - m1)
        mix = (sel1 + e2 * sel2) / (1.0 + e2)
        out_ref[...] = bga_s[:, 0:1] + bga_s[:, 1:2] * mix


def _run_heads(numv, catc, w1ut, big, gw1t, gvec, rw1t, rvec, rw2t, aw1t,
               sc3, rbv, eb2s, ew1t, eb1, ew2):
    bb = 256
    nb = BATCH // bb
    grid = (nb, NEXP)
    c = lambda i, e: (0, 0)
    out = pl.pallas_call(
        _heads_kernel,
        grid=grid,
        in_specs=[
            pl.BlockSpec((bb, HP), lambda i, e: (i, 0)),
            pl.BlockSpec((bb, HP), lambda i, e: (i, 0)),
            pl.BlockSpec((HP, D), c),
            pl.BlockSpec((4, D), c),
            pl.BlockSpec((D, MID), c),
            pl.BlockSpec((2, MID), c),
            pl.BlockSpec((D, RH), c),
            pl.BlockSpec((3, RH), c),
            pl.BlockSpec((RH, NEXP), c),
            pl.BlockSpec((D, RH), c),
            pl.BlockSpec(memory_space=pltpu.SMEM),
            pl.BlockSpec((1, NEXP), c),
            pl.BlockSpec(memory_space=pltpu.SMEM),
            pl.BlockSpec((1, D, MID), lambda i, e: (e, 0, 0)),
            pl.BlockSpec((1, 1, MID), lambda i, e: (e, 0, 0)),
            pl.BlockSpec((1, 1, MID), lambda i, e: (e, 0, 0)),
        ],
        out_specs=pl.BlockSpec((bb, 1), lambda i, e: (i, 0)),
        out_shape=jax.ShapeDtypeStruct((BATCH, 1), jnp.float32),
        scratch_shapes=[
            pltpu.VMEM((bb, D), jnp.float32),
            pltpu.VMEM((bb, NEXP), jnp.float32),
            pltpu.VMEM((bb, NEXP), jnp.float32),
            pltpu.VMEM((bb, NEXP), jnp.float32),
        ],
    )(numv, catc, w1ut, big, gw1t, gvec, rw1t, rvec, rw2t, aw1t, sc3, rbv,
      eb2s, ew1t, eb1[:, None, :], ew2[:, None, :])
    return out[:, 0]


def _padlane(x, n):
    return jnp.pad(x, ((0, 0),) * (x.ndim - 1) + ((0, n - x.shape[-1]),))


def kernel(x_num, x_cat, params):
    p = params
    g = p['ln_g']
    lb = p['ln_b']
    W0 = p['W0']
    b0 = p['b0']
    W0v = W0[H:]
    b0v = b0[H:]
    sgu_g = p['sgu_ln_g']
    sgu_b = p['sgu_ln_b']
    sguw = p['sgu_W'][0]

    def lnv(x, gg, bb):
        m = x.mean(-1, keepdims=True)
        v = x.var(-1, keepdims=True)
        return (x - m) / jnp.sqrt(v + EPS) * gg + bb

    row0 = p['tok_weight'][0]
    ln0 = lnv(row0[None], g, lb)[0]
    z0 = ln0 @ W0.T + b0
    u0 = _gelu(z0[:H])
    vln0 = lnv(_gelu(z0[H:])[None], sgu_g, sgu_b)[0]
    const = (sguw[0] * vln0 + p['sgu_b'][0]
             + jnp.sum(sguw[1:NNUM + 1]) * sgu_b)

    wmat = p['tok_weight'][1:]
    bmat = p['tok_bias'][:NNUM]
    a1 = _padlane((wmat * g) @ W0v.T, HP)
    a2 = _padlane((bmat * g) @ W0v.T, HP)
    a3 = _padlane((g @ W0v.T)[None], HP)
    a4 = _padlane((lb @ W0v.T + b0v)[None], HP)
    a34 = jnp.concatenate([a3, a4], axis=0)
    mw = wmat.mean(1)
    mb = bmat.mean(1)
    vw = wmat.var(1)
    vb = bmat.var(1)
    cwb = (wmat * bmat).mean(1) - mw * mb
    mom = jnp.stack([mw, mb, vw, cwb, vb, sguw[1:NNUM + 1],
                     jnp.zeros_like(mw), jnp.zeros_like(mw)])
    wg = _padlane(sguw[1:NNUM + 1, None] * sgu_g[None], HP)

    w0vt = _padlane(W0v.T, HP)
    b0vp = _padlane(b0v[None], HP)
    sgugb = jnp.concatenate([_padlane(sgu_g[None], HP),
                             _padlane(sgu_b[None], HP)], axis=0)
    scal = sguw[NNUM + 1:].astype(jnp.float32)
    bias8 = p['tok_bias'][NNUM:]

    table = _build_table(p['cat_emb'], bias8, g[None], lb[None], w0vt, b0vp,
                         sgugb, scal)

    fidx = (x_cat.astype(jnp.int32)
            + (jnp.arange(NCAT, dtype=jnp.int32) * CARD)[None]).reshape(-1)
    catc = _gather_fn(table, fidx)

    numv = _run_numeric(x_num.astype(jnp.float32), a1, a2, a34, wg, mom,
                        _padlane(const[None], HP))

    W1u = p['W1'] * u0[None]
    w1ut = _padlane(W1u, HP).T
    big = jnp.stack([p['b1'] + row0, g, lb, p['base_W'][0]])
    gvec = jnp.stack([p['gb1'], p['gW2'][0]])
    rvec = jnp.stack([p['rb1'], p['ab1'], p['aW2'][0]])
    sc3 = jnp.concatenate([p['base_b'], p['gb2'], p['ab2'],
                           jnp.zeros((5,), jnp.float32)])
    rbv = p['rb2'][None]
    eb2s = p['eb2']
    ew1t = p['eW1'].transpose(0, 2, 1)

    return _run_heads(numv, catc, w1ut, big, p['gW1'].T, gvec, p['rW1'].T,
                      rvec, p['rW2'].T, p['aW1'].T, sc3, rbv, eb2s, ew1t,
                      p['eb1'], p['eW2'])


# speedup vs baseline: 1.0506x; 1.0506x over previous
import functools
import jax
import jax.numpy as jnp
from jax import lax
from jax.experimental import pallas as pl
from jax.experimental.pallas import tpu as pltpu
from jax.experimental.pallas import tpu_sc as plsc

BATCH = 1024
D = 1024
H = 675
HP = 768
NNUM = 32
NCAT = 8
CARD = 1000
NEXP = 8
MID = 512
RH = 256
EPS = 1e-5
_SQRT1_2 = 0.7071067811865476

NC, NS = 2, 16
NW = NC * NS
SPW = BATCH // NW
ROWS_PW = SPW * NCAT
CH_ROWS = 32
NCHUNK = ROWS_PW // CH_ROWS
SPC = CH_ROWS // NCAT
LGRP = HP // 16


def _gelu(x):
    return 0.5 * x * (1.0 + lax.erf(x * _SQRT1_2))


def _table_kernel(emb_ref, bias_ref, lng_ref, lnb_ref, w0vt_ref, b0v_ref,
                  sgugb_ref, scal_ref, out_ref):
    j = pl.program_id(0) // 5
    x = emb_ref[...] + bias_ref[0]
    m = jnp.mean(x, axis=-1, keepdims=True)
    v = jnp.mean((x - m) * (x - m), axis=-1, keepdims=True)
    y = (x - m) * lax.rsqrt(v + EPS) * lng_ref[...] + lnb_ref[...]
    z = lax.dot_general(y, w0vt_ref[...], (((1,), (1,)), ((), ())),
                        preferred_element_type=jnp.float32) + b0v_ref[...]
    h = _gelu(z)
    mh = jnp.sum(h, axis=-1, keepdims=True) * (1.0 / H)
    hc = h - mh
    vh = (jnp.sum(hc * hc, axis=-1, keepdims=True)
          - (HP - H) * mh * mh) * (1.0 / H)
    yv = hc * lax.rsqrt(vh + EPS) * sgugb_ref[0:1, :] + sgugb_ref[1:2, :]
    out_ref[...] = yv * scal_ref[j]


def _build_table(cat_emb, bias8, lng, lnb, w0vt, b0v, sgugb, scal):
    grid = 40
    rb = 200
    return pl.pallas_call(
        _table_kernel,
        grid=(grid,),
        in_specs=[
            pl.BlockSpec((rb, D), lambda i: (i, 0)),
            pl.BlockSpec((1, 1, D), lambda i: (i // 5, 0, 0)),
            pl.BlockSpec((1, D), lambda i: (0, 0)),
            pl.BlockSpec((1, D), lambda i: (0, 0)),
            pl.BlockSpec((HP, D), lambda i: (0, 0)),
            pl.BlockSpec((1, HP), lambda i: (0, 0)),
            pl.BlockSpec((2, HP), lambda i: (0, 0)),
            pl.BlockSpec(memory_space=pltpu.SMEM),
        ],
        out_specs=pl.BlockSpec((rb, HP), lambda i: (i, 0)),
        out_shape=jax.ShapeDtypeStruct((NCAT * CARD, HP), jnp.float32),
    )(cat_emb, bias8[:, None, :], lng, lnb, w0vt, b0v, sgugb, scal)


def _make_gather():
    mesh = plsc.VectorSubcoreMesh(core_axis_name="c", subcore_axis_name="s")

    @functools.partial(
        pl.kernel, mesh=mesh,
        out_type=jax.ShapeDtypeStruct((BATCH, HP), jnp.float32),
        scratch_types=[
            pltpu.VMEM((ROWS_PW,), jnp.int32),
            pltpu.VMEM((CH_ROWS, HP), jnp.float32),
            pltpu.VMEM((CH_ROWS, HP), jnp.float32),
            pltpu.VMEM((SPW, HP), jnp.float32),
            pltpu.SemaphoreType.DMA,
            pltpu.SemaphoreType.DMA,
        ],
    )
    def gather(table_hbm, fidx_hbm, out_hbm, idx_v, bufa, bufb, out_v, sema, semb):
        wid = lax.axis_index("s") * NC + lax.axis_index("c")
        base = wid * SPW
        pltpu.sync_copy(fidx_hbm.at[pl.ds(base * NCAT, ROWS_PW)], idx_v)
        bufs = (bufa, bufb)
        sems = (sema, semb)
        pltpu.make_async_copy(
            table_hbm.at[idx_v.at[pl.ds(0, CH_ROWS)]], bufs[0], sems[0]).start()
        for c in range(NCHUNK):
            if c + 1 < NCHUNK:
                pltpu.make_async_copy(
                    table_hbm.at[idx_v.at[pl.ds((c + 1) * CH_ROWS, CH_ROWS)]],
                    bufs[(c + 1) % 2], sems[(c + 1) % 2]).start()
            buf = bufs[c % 2]
            pltpu.make_async_copy(
                table_hbm.at[idx_v.at[pl.ds(c * CH_ROWS, CH_ROWS)]],
                buf, sems[c % 2]).wait()
            for s in range(SPC):
                r0 = s * NCAT
                orow = c * SPC + s

                def grp(gi, _):
                    sl = pl.ds(gi * 16, 16)
                    acc0 = buf[r0 + 0, sl] + buf[r0 + 1, sl]
                    acc1 = buf[r0 + 2, sl] + buf[r0 + 3, sl]
                    acc2 = buf[r0 + 4, sl] + buf[r0 + 5, sl]
                    acc3 = buf[r0 + 6, sl] + buf[r0 + 7, sl]
                    out_v[orow, sl] = (acc0 + acc1) + (acc2 + acc3)
                    return 0

                lax.fori_loop(0, LGRP, grp, 0)
        pltpu.sync_copy(out_v, out_hbm.at[pl.ds(base, SPW)])

    return gather


def _gather_fn(table, fidx):
    return _make_gather()(table, fidx)


def _numeric_kernel(xnum_ref, a1_ref, a2_ref, a34_ref, wg_ref, mom_ref,
                    constv_ref, out_ref):
    acc = jnp.broadcast_to(constv_ref[...], out_ref.shape)
    s_all = xnum_ref[...]
    a3 = a34_ref[0:1, :]
    a4 = a34_ref[1:2, :]
    for t in range(NNUM):
        s = s_all[:, t:t + 1]
        m = s * mom_ref[0, t] + mom_ref[1, t]
        var = s * s * mom_ref[2, t] + 2.0 * s * mom_ref[3, t] + mom_ref[4, t]
        rsinv = lax.rsqrt(var + EPS)
        z = (s * a1_ref[t:t + 1, :] + a2_ref[t:t + 1, :] - m * a3) * rsinv + a4
        h = _gelu(z)
        mh = jnp.sum(h, axis=-1, keepdims=True) * (1.0 / H)
        hc = h - mh
        vh = (jnp.sum(hc * hc, axis=-1, keepdims=True)
              - (HP - H) * mh * mh) * (1.0 / H)
        acc = acc + hc * lax.rsqrt(vh + EPS) * wg_ref[t:t + 1, :]
    out_ref[...] = acc


def _run_numeric(xnum, a1, a2, a34, wg, mom, constv):
    bb = 256
    c = lambda i: (0, 0)
    return pl.pallas_call(
        _numeric_kernel,
        grid=(BATCH // bb,),
        in_specs=[
            pl.BlockSpec((bb, NNUM), lambda i: (i, 0)),
            pl.BlockSpec((NNUM, HP), c),
            pl.BlockSpec((NNUM, HP), c),
            pl.BlockSpec((2, HP), c),
            pl.BlockSpec((NNUM, HP), c),
            pl.BlockSpec(memory_space=pltpu.SMEM),
            pl.BlockSpec((1, HP), c),
        ],
        out_specs=pl.BlockSpec((bb, HP), lambda i: (i, 0)),
        out_shape=jax.ShapeDtypeStruct((BATCH, HP), jnp.float32),
    )(xnum, a1, a2, a34, wg, mom, constv)


def _dgt(x, w, prec=None):
    return lax.dot_general(x, w, (((1,), (1,)), ((), ())),
                           preferred_element_type=jnp.float32, precision=prec)


def _heads_kernel(numv_ref, catc_ref, u0_ref, w1p_ref, x0b_ref, lng_ref,
                  lnb_ref, bw_ref, gw1_ref, gb1_ref, gw2_ref, rw1_ref,
                  rb1_ref, rw2_ref, aw1_ref, ab1_ref, aw2_ref, sc3_ref,
                  rbv_ref, eb2s_ref, ew1_ref, eb1_ref, ew2_ref, out_ref,
                  hid_s, rl_s, eo_s, bga_s):
    e = pl.program_id(1)

    @pl.when(e == 0)
    def _first():
        vu = (catc_ref[...] + numv_ref[...]) * u0_ref[...]
        x0 = _dgt(vu, w1p_ref[...], lax.Precision.HIGHEST) + x0b_ref[...]
        m0 = jnp.mean(x0, axis=-1, keepdims=True)
        v0 = jnp.mean((x0 - m0) * (x0 - m0), axis=-1, keepdims=True)
        hid = (x0 - m0) * lax.rsqrt(v0 + EPS) * lng_ref[...] + lnb_ref[...]
        hid_s[...] = hid
        base = jnp.sum(hid * bw_ref[...], axis=-1, keepdims=True) + sc3_ref[0]
        gr = jnp.maximum(_dgt(hid, gw1_ref[...]) + gb1_ref[...], 0.0)
        gout = jnp.sum(gr * gw2_ref[...], axis=-1, keepdims=True) + sc3_ref[1]
        rr = jnp.maximum(_dgt(hid, rw1_ref[...]) + rb1_ref[...], 0.0)
        rl_s[...] = _dgt(rr, rw2_ref[...]) + rbv_ref[...]
        ar = jnp.maximum(_dgt(hid, aw1_ref[...]) + ab1_ref[...], 0.0)
        av = jnp.sum(ar * aw2_ref[...], axis=-1, keepdims=True) + sc3_ref[2]
        alpha = 1.0 / (1.0 + jnp.exp(-av))
        bga_s[...] = jnp.concatenate(
            [base + gout, alpha, jnp.zeros_like(base), jnp.zeros_like(base),
             base, base, base, base], axis=1)
        eo_s[...] = jnp.zeros(eo_s.shape, eo_s.dtype)

    eh = jnp.maximum(_dgt(hid_s[...], ew1_ref[0]) + eb1_ref[0], 0.0)
    eoe = jnp.sum(eh * ew2_ref[0], axis=-1, keepdims=True) + eb2s_ref[e]
    onehot = (lax.broadcasted_iota(jnp.int32, eo_s.shape, 1) == e
              ).astype(jnp.float32)
    eo_s[...] += eoe * onehot

    @pl.when(e == NEXP - 1)
    def _last():
        rl = rl_s[...]
        eo = eo_s[...]
        tri = (lax.broadcasted_iota(jnp.int32, (NEXP, NEXP), 0)
               <= lax.broadcasted_iota(jnp.int32, (NEXP, NEXP), 1)
               ).astype(jnp.float32)
        m1 = jnp.max(rl, axis=-1, keepdims=True)
        eq1 = (rl == m1).astype(jnp.float32)
        f1 = eq1 * (jnp.dot(eq1, tri, preferred_element_type=jnp.float32)
                    == 1.0).astype(jnp.float32)
        rl2 = jnp.where(f1 > 0.0, -jnp.inf, rl)
        m2 = jnp.max(rl2, axis=-1, keepdims=True)
        eq2 = (rl2 == m2).astype(jnp.float32)
        f2 = eq2 * (jnp.dot(eq2, tri, preferred_element_type=jnp.float32)
                    == 1.0).astype(jnp.float32)
        sel1 = jnp.sum(eo * f1, axis=-1, keepdims=True)
        sel2 = jnp.sum(eo * f2, axis=-1, keepdims=True)
        e2 = jnp.exp(m2 - m1)
        mix = (sel1 + e2 * sel2) / (1.0 + e2)
        out_ref[...] = bga_s[:, 0:1] + bga_s[:, 1:2] * mix


def _run_heads(numv, catc, u0p, w1p, x0b, lng, lnb, bw, gw1, gb1, gw2, rw1,
               rb1, rw2, aw1, ab1, aw2, sc3, rbv, eb2s, ew1, eb1, ew2):
    bb = 256
    nb = BATCH // bb
    grid = (nb, NEXP)
    c = lambda i, e: (0, 0)
    out = pl.pallas_call(
        _heads_kernel,
        grid=grid,
        in_specs=[
            pl.BlockSpec((bb, HP), lambda i, e: (i, 0)),
            pl.BlockSpec((bb, HP), lambda i, e: (i, 0)),
            pl.BlockSpec((1, HP), c),
            pl.BlockSpec((D, HP), c),
            pl.BlockSpec((1, D), c),
            pl.BlockSpec((1, D), c),
            pl.BlockSpec((1, D), c),
            pl.BlockSpec((1, D), c),
            pl.BlockSpec((MID, D), c),
            pl.BlockSpec((1, MID), c),
            pl.BlockSpec((1, MID), c),
            pl.BlockSpec((RH, D), c),
            pl.BlockSpec((1, RH), c),
            pl.BlockSpec((NEXP, RH), c),
            pl.BlockSpec((RH, D), c),
            pl.BlockSpec((1, RH), c),
            pl.BlockSpec((1, RH), c),
            pl.BlockSpec(memory_space=pltpu.SMEM),
            pl.BlockSpec((1, NEXP), c),
            pl.BlockSpec(memory_space=pltpu.SMEM),
            pl.BlockSpec((1, MID, D), lambda i, e: (e, 0, 0)),
            pl.BlockSpec((1, 1, MID), lambda i, e: (e, 0, 0)),
            pl.BlockSpec((1, 1, MID), lambda i, e: (e, 0, 0)),
        ],
        out_specs=pl.BlockSpec((bb, 1), lambda i, e: (i, 0)),
        out_shape=jax.ShapeDtypeStruct((BATCH, 1), jnp.float32),
        scratch_shapes=[
            pltpu.VMEM((bb, D), jnp.float32),
            pltpu.VMEM((bb, NEXP), jnp.float32),
            pltpu.VMEM((bb, NEXP), jnp.float32),
            pltpu.VMEM((bb, NEXP), jnp.float32),
        ],
    )(numv, catc, u0p, w1p, x0b, lng, lnb, bw, gw1, gb1, gw2, rw1, rb1, rw2,
      aw1, ab1, aw2, sc3, rbv, eb2s, ew1, eb1[:, None, :], ew2[:, None, :])
    return out[:, 0]


def _padlane(x, n):
    return jnp.pad(x, ((0, 0),) * (x.ndim - 1) + ((0, n - x.shape[-1]),))


def kernel(x_num, x_cat, params):
    p = params
    g = p['ln_g']
    lb = p['ln_b']
    W0 = p['W0']
    b0 = p['b0']
    W0v = W0[H:]
    b0v = b0[H:]
    sgu_g = p['sgu_ln_g']
    sgu_b = p['sgu_ln_b']
    sguw = p['sgu_W'][0]

    def lnv(x, gg, bb):
        m = x.mean(-1, keepdims=True)
        v = x.var(-1, keepdims=True)
        return (x - m) / jnp.sqrt(v + EPS) * gg + bb

    row0 = p['tok_weight'][0]
    ln0 = lnv(row0[None], g, lb)[0]
    z0 = ln0 @ W0.T + b0
    u0 = _gelu(z0[:H])
    vln0 = lnv(_gelu(z0[H:])[None], sgu_g, sgu_b)[0]
    const = (sguw[0] * vln0 + p['sgu_b'][0]
             + jnp.sum(sguw[1:NNUM + 1]) * sgu_b)

    wmat = p['tok_weight'][1:]
    bmat = p['tok_bias'][:NNUM]
    a1 = _padlane((wmat * g) @ W0v.T, HP)
    a2 = _padlane((bmat * g) @ W0v.T, HP)
    a3 = _padlane((g @ W0v.T)[None], HP)
    a4 = _padlane((lb @ W0v.T + b0v)[None], HP)
    a34 = jnp.concatenate([a3, a4], axis=0)
    mw = wmat.mean(1)
    mb = bmat.mean(1)
    vw = wmat.var(1)
    vb = bmat.var(1)
    cwb = (wmat * bmat).mean(1) - mw * mb
    mom = jnp.stack([mw, mb, vw, cwb, vb, sguw[1:NNUM + 1],
                     jnp.zeros_like(mw), jnp.zeros_like(mw)])
    wg = _padlane(sguw[1:NNUM + 1, None] * sgu_g[None], HP)

    w0vp = jnp.pad(W0v, ((0, HP - H), (0, 0)))
    b0vp = _padlane(b0v[None], HP)
    sgugb = jnp.concatenate([_padlane(sgu_g[None], HP),
                             _padlane(sgu_b[None], HP)], axis=0)
    scal = sguw[NNUM + 1:].astype(jnp.float32)
    bias8 = p['tok_bias'][NNUM:]

    numv = _run_numeric(x_num.astype(jnp.float32), a1, a2, a34, wg, mom,
                        _padlane(const[None], HP))

    table = _build_table(p['cat_emb'], bias8, g[None], lb[None], w0vp, b0vp,
                         sgugb, scal)

    fidx = (x_cat.astype(jnp.int32)
            + (jnp.arange(NCAT, dtype=jnp.int32) * CARD)[None]).reshape(-1)
    catc = _gather_fn(table, fidx)

    u0p = _padlane(u0[None], HP)
    w1p = _padlane(p['W1'], HP)
    sc3 = jnp.concatenate([p['base_b'], p['gb2'], p['ab2'],
                           jnp.zeros((5,), jnp.float32)])

    return _run_heads(numv, catc, u0p, w1p, (p['b1'] + row0)[None], g[None],
                      lb[None], p['base_W'], p['gW1'], p['gb1'][None],
                      p['gW2'], p['rW1'], p['rb1'][None], p['rW2'],
                      p['aW1'], p['ab1'][None], p['aW2'], sc3,
                      p['rb2'][None], p['eb2'], p['eW1'], p['eb1'], p['eW2'])


# speedup vs baseline: 1.0686x; 1.0171x over previous
import functools
import jax
import jax.numpy as jnp
from jax import lax
from jax.experimental import pallas as pl
from jax.experimental.pallas import tpu as pltpu
from jax.experimental.pallas import tpu_sc as plsc

BATCH = 1024
D = 1024
H = 675
HP = 768
NNUM = 32
NCAT = 8
CARD = 1000
NEXP = 8
MID = 512
RH = 256
EPS = 1e-5
_SQRT1_2 = 0.7071067811865476

NC, NS = 2, 16
NW = NC * NS
SPW = BATCH // NW
ROWS_PW = SPW * NCAT
CH_ROWS = 32
NCHUNK = ROWS_PW // CH_ROWS
SPC = CH_ROWS // NCAT
LGRP = HP // 16


def _gelu(x):
    return 0.5 * x * (1.0 + lax.erf(x * _SQRT1_2))


def _table_kernel(emb_ref, bias_ref, lng_ref, lnb_ref, w0vt_ref, b0v_ref,
                  sgugb_ref, scal_ref, out_ref):
    j = pl.program_id(0) // 5
    x = emb_ref[...] + bias_ref[0]
    m = jnp.mean(x, axis=-1, keepdims=True)
    v = jnp.mean((x - m) * (x - m), axis=-1, keepdims=True)
    y = (x - m) * lax.rsqrt(v + EPS) * lng_ref[...] + lnb_ref[...]
    z = lax.dot_general(y, w0vt_ref[...], (((1,), (1,)), ((), ())),
                        preferred_element_type=jnp.float32) + b0v_ref[...]
    h = _gelu(z)
    mh = jnp.sum(h, axis=-1, keepdims=True) * (1.0 / H)
    hc = h - mh
    vh = (jnp.sum(hc * hc, axis=-1, keepdims=True)
          - (HP - H) * mh * mh) * (1.0 / H)
    yv = hc * lax.rsqrt(vh + EPS) * sgugb_ref[0:1, :] + sgugb_ref[1:2, :]
    out_ref[...] = yv * scal_ref[j]


def _build_table(cat_emb, bias8, lng, lnb, w0vt, b0v, sgugb, scal):
    grid = 40
    rb = 200
    return pl.pallas_call(
        _table_kernel,
        grid=(grid,),
        in_specs=[
            pl.BlockSpec((rb, D), lambda i: (i, 0)),
            pl.BlockSpec((1, 1, D), lambda i: (i // 5, 0, 0)),
            pl.BlockSpec((1, D), lambda i: (0, 0)),
            pl.BlockSpec((1, D), lambda i: (0, 0)),
            pl.BlockSpec((HP, D), lambda i: (0, 0)),
            pl.BlockSpec((1, HP), lambda i: (0, 0)),
            pl.BlockSpec((2, HP), lambda i: (0, 0)),
            pl.BlockSpec(memory_space=pltpu.SMEM),
        ],
        out_specs=pl.BlockSpec((rb, HP), lambda i: (i, 0)),
        out_shape=jax.ShapeDtypeStruct((NCAT * CARD, HP), jnp.float32),
    )(cat_emb, bias8[:, None, :], lng, lnb, w0vt, b0v, sgugb, scal)


def _make_gather():
    mesh = plsc.VectorSubcoreMesh(core_axis_name="c", subcore_axis_name="s")

    @functools.partial(
        pl.kernel, mesh=mesh,
        out_type=jax.ShapeDtypeStruct((BATCH, HP), jnp.float32),
        scratch_types=[
            pltpu.VMEM((ROWS_PW,), jnp.int32),
            pltpu.VMEM((CH_ROWS, HP), jnp.float32),
            pltpu.VMEM((CH_ROWS, HP), jnp.float32),
            pltpu.VMEM((SPW, HP), jnp.float32),
            pltpu.SemaphoreType.DMA,
            pltpu.SemaphoreType.DMA,
        ],
    )
    def gather(table_hbm, fidx_hbm, out_hbm, idx_v, bufa, bufb, out_v, sema, semb):
        wid = lax.axis_index("s") * NC + lax.axis_index("c")
        base = wid * SPW
        pltpu.sync_copy(fidx_hbm.at[pl.ds(base * NCAT, ROWS_PW)], idx_v)
        bufs = (bufa, bufb)
        sems = (sema, semb)
        pltpu.make_async_copy(
            table_hbm.at[idx_v.at[pl.ds(0, CH_ROWS)]], bufs[0], sems[0]).start()
        for c in range(NCHUNK):
            if c + 1 < NCHUNK:
                pltpu.make_async_copy(
                    table_hbm.at[idx_v.at[pl.ds((c + 1) * CH_ROWS, CH_ROWS)]],
                    bufs[(c + 1) % 2], sems[(c + 1) % 2]).start()
            buf = bufs[c % 2]
            pltpu.make_async_copy(
                table_hbm.at[idx_v.at[pl.ds(c * CH_ROWS, CH_ROWS)]],
                buf, sems[c % 2]).wait()
            for s in range(SPC):
                r0 = s * NCAT
                orow = c * SPC + s

                def grp(gi, _):
                    sl = pl.ds(gi * 16, 16)
                    acc0 = buf[r0 + 0, sl] + buf[r0 + 1, sl]
                    acc1 = buf[r0 + 2, sl] + buf[r0 + 3, sl]
                    acc2 = buf[r0 + 4, sl] + buf[r0 + 5, sl]
                    acc3 = buf[r0 + 6, sl] + buf[r0 + 7, sl]
                    out_v[orow, sl] = (acc0 + acc1) + (acc2 + acc3)
                    return 0

                lax.fori_loop(0, LGRP, grp, 0)
        pltpu.sync_copy(out_v, out_hbm.at[pl.ds(base, SPW)])

    return gather


def _gather_fn(table, fidx):
    return _make_gather()(table, fidx)


def _numeric_kernel(xnum_ref, a1_ref, a2_ref, a34_ref, wg_ref, mom_ref,
                    constv_ref, out_ref):
    acc = jnp.broadcast_to(constv_ref[...], out_ref.shape)
    s_all = xnum_ref[...]
    a3 = a34_ref[0:1, :]
    a4 = a34_ref[1:2, :]
    for t in range(NNUM):
        s = s_all[:, t:t + 1]
        m = s * mom_ref[0, t] + mom_ref[1, t]
        var = s * s * mom_ref[2, t] + 2.0 * s * mom_ref[3, t] + mom_ref[4, t]
        rsinv = lax.rsqrt(var + EPS)
        z = (s * a1_ref[t:t + 1, :] + a2_ref[t:t + 1, :] - m * a3) * rsinv + a4
        h = _gelu(z)
        mh = jnp.sum(h, axis=-1, keepdims=True) * (1.0 / H)
        hc = h - mh
        vh = (jnp.sum(hc * hc, axis=-1, keepdims=True)
              - (HP - H) * mh * mh) * (1.0 / H)
        acc = acc + hc * lax.rsqrt(vh + EPS) * wg_ref[t:t + 1, :]
    out_ref[...] = acc


def _run_numeric(xnum, a1, a2, a34, wg, mom, constv):
    bb = 256
    c = lambda i: (0, 0)
    return pl.pallas_call(
        _numeric_kernel,
        grid=(BATCH // bb,),
        in_specs=[
            pl.BlockSpec((bb, NNUM), lambda i: (i, 0)),
            pl.BlockSpec((NNUM, HP), c),
            pl.BlockSpec((NNUM, HP), c),
            pl.BlockSpec((2, HP), c),
            pl.BlockSpec((NNUM, HP), c),
            pl.BlockSpec(memory_space=pltpu.SMEM),
            pl.BlockSpec((1, HP), c),
        ],
        out_specs=pl.BlockSpec((bb, HP), lambda i: (i, 0)),
        out_shape=jax.ShapeDtypeStruct((BATCH, HP), jnp.float32),
    )(xnum, a1, a2, a34, wg, mom, constv)


def _dgt(x, w, prec=None):
    return lax.dot_general(x, w, (((1,), (1,)), ((), ())),
                           preferred_element_type=jnp.float32, precision=prec)


def _heads_kernel(numv_ref, catc_ref, u0_ref, w1p_ref, x0b_ref, lng_ref,
                  lnb_ref, bw_ref, gw1_ref, gb1_ref, gw2_ref, rw1_ref,
                  rb1_ref, rw2_ref, aw1_ref, ab1_ref, aw2_ref, sc3_ref,
                  rbv_ref, eb2s_ref, ew1_ref, eb1_ref, ew2_ref, out_ref,
                  hid_s, rl_s, eo_s, bga_s):
    e = pl.program_id(1)

    @pl.when(e == 0)
    def _first():
        vu = (catc_ref[...] + numv_ref[...]) * u0_ref[...]
        x0 = jnp.dot(vu, w1p_ref[...], preferred_element_type=jnp.float32,
                     precision=lax.Precision.HIGHEST) + x0b_ref[...]
        m0 = jnp.mean(x0, axis=-1, keepdims=True)
        v0 = jnp.mean((x0 - m0) * (x0 - m0), axis=-1, keepdims=True)
        hid = (x0 - m0) * lax.rsqrt(v0 + EPS) * lng_ref[...] + lnb_ref[...]
        hid_s[...] = hid
        base = jnp.sum(hid * bw_ref[...], axis=-1, keepdims=True) + sc3_ref[0]
        gr = jnp.maximum(_dgt(hid, gw1_ref[...]) + gb1_ref[...], 0.0)
        gout = jnp.sum(gr * gw2_ref[...], axis=-1, keepdims=True) + sc3_ref[1]
        rr = jnp.maximum(_dgt(hid, rw1_ref[...]) + rb1_ref[...], 0.0)
        rl_s[...] = _dgt(rr, rw2_ref[...]) + rbv_ref[...]
        ar = jnp.maximum(_dgt(hid, aw1_ref[...]) + ab1_ref[...], 0.0)
        av = jnp.sum(ar * aw2_ref[...], axis=-1, keepdims=True) + sc3_ref[2]
        alpha = 1.0 / (1.0 + jnp.exp(-av))
        bga_s[...] = jnp.concatenate(
            [base + gout, alpha, jnp.zeros_like(base), jnp.zeros_like(base),
             base, base, base, base], axis=1)
        eo_s[...] = jnp.zeros(eo_s.shape, eo_s.dtype)

    eh = jnp.maximum(_dgt(hid_s[...], ew1_ref[0]) + eb1_ref[0], 0.0)
    eoe = jnp.sum(eh * ew2_ref[0], axis=-1, keepdims=True) + eb2s_ref[e]
    onehot = (lax.broadcasted_iota(jnp.int32, eo_s.shape, 1) == e
              ).astype(jnp.float32)
    eo_s[...] += eoe * onehot

    @pl.when(e == NEXP - 1)
    def _last():
        rl = rl_s[...]
        eo = eo_s[...]
        tri = (lax.broadcasted_iota(jnp.int32, (NEXP, NEXP), 0)
               <= lax.broadcasted_iota(jnp.int32, (NEXP, NEXP), 1)
               ).astype(jnp.float32)
        m1 = jnp.max(rl, axis=-1, keepdims=True)
        eq1 = (rl == m1).astype(jnp.float32)
        f1 = eq1 * (jnp.dot(eq1, tri, preferred_element_type=jnp.float32)
                    == 1.0).astype(jnp.float32)
        rl2 = jnp.where(f1 > 0.0, -jnp.inf, rl)
        m2 = jnp.max(rl2, axis=-1, keepdims=True)
        eq2 = (rl2 == m2).astype(jnp.float32)
        f2 = eq2 * (jnp.dot(eq2, tri, preferred_element_type=jnp.float32)
                    == 1.0).astype(jnp.float32)
        sel1 = jnp.sum(eo * f1, axis=-1, keepdims=True)
        sel2 = jnp.sum(eo * f2, axis=-1, keepdims=True)
        e2 = jnp.exp(m2 - m1)
        mix = (sel1 + e2 * sel2) / (1.0 + e2)
        out_ref[...] = bga_s[:, 0:1] + bga_s[:, 1:2] * mix


def _run_heads(numv, catc, u0p, w1p, x0b, lng, lnb, bw, gw1, gb1, gw2, rw1,
               rb1, rw2, aw1, ab1, aw2, sc3, rbv, eb2s, ew1, eb1, ew2):
    bb = 256
    nb = BATCH // bb
    grid = (nb, NEXP)
    c = lambda i, e: (0, 0)
    out = pl.pallas_call(
        _heads_kernel,
        grid=grid,
        in_specs=[
            pl.BlockSpec((bb, HP), lambda i, e: (i, 0)),
            pl.BlockSpec((bb, HP), lambda i, e: (i, 0)),
            pl.BlockSpec((1, HP), c),
            pl.BlockSpec((HP, D), c),
            pl.BlockSpec((1, D), c),
            pl.BlockSpec((1, D), c),
            pl.BlockSpec((1, D), c),
            pl.BlockSpec((1, D), c),
            pl.BlockSpec((MID, D), c),
            pl.BlockSpec((1, MID), c),
            pl.BlockSpec((1, MID), c),
            pl.BlockSpec((RH, D), c),
            pl.BlockSpec((1, RH), c),
            pl.BlockSpec((NEXP, RH), c),
            pl.BlockSpec((RH, D), c),
            pl.BlockSpec((1, RH), c),
            pl.BlockSpec((1, RH), c),
            pl.BlockSpec(memory_space=pltpu.SMEM),
            pl.BlockSpec((1, NEXP), c),
            pl.BlockSpec(memory_space=pltpu.SMEM),
            pl.BlockSpec((1, MID, D), lambda i, e: (e, 0, 0)),
            pl.BlockSpec((1, 1, MID), lambda i, e: (e, 0, 0)),
            pl.BlockSpec((1, 1, MID), lambda i, e: (e, 0, 0)),
        ],
        out_specs=pl.BlockSpec((bb, 1), lambda i, e: (i, 0)),
        out_shape=jax.ShapeDtypeStruct((BATCH, 1), jnp.float32),
        scratch_shapes=[
            pltpu.VMEM((bb, D), jnp.float32),
            pltpu.VMEM((bb, NEXP), jnp.float32),
            pltpu.VMEM((bb, NEXP), jnp.float32),
            pltpu.VMEM((bb, NEXP), jnp.float32),
        ],
    )(numv, catc, u0p, w1p, x0b, lng, lnb, bw, gw1, gb1, gw2, rw1, rb1, rw2,
      aw1, ab1, aw2, sc3, rbv, eb2s, ew1, eb1[:, None, :], ew2[:, None, :])
    return out[:, 0]


def _padlane(x, n):
    return jnp.pad(x, ((0, 0),) * (x.ndim - 1) + ((0, n - x.shape[-1]),))


def kernel(x_num, x_cat, params):
    p = params
    g = p['ln_g']
    lb = p['ln_b']
    W0 = p['W0']
    b0 = p['b0']
    W0v = W0[H:]
    b0v = b0[H:]
    sgu_g = p['sgu_ln_g']
    sgu_b = p['sgu_ln_b']
    sguw = p['sgu_W'][0]

    def lnv(x, gg, bb):
        m = x.mean(-1, keepdims=True)
        v = x.var(-1, keepdims=True)
        return (x - m) / jnp.sqrt(v + EPS) * gg + bb

    row0 = p['tok_weight'][0]
    ln0 = lnv(row0[None], g, lb)[0]
    z0 = ln0 @ W0.T + b0
    u0 = _gelu(z0[:H])
    vln0 = lnv(_gelu(z0[H:])[None], sgu_g, sgu_b)[0]
    const = (sguw[0] * vln0 + p['sgu_b'][0]
             + jnp.sum(sguw[1:NNUM + 1]) * sgu_b)

    wmat = p['tok_weight'][1:]
    bmat = p['tok_bias'][:NNUM]
    a1 = _padlane((wmat * g) @ W0v.T, HP)
    a2 = _padlane((bmat * g) @ W0v.T, HP)
    a3 = _padlane((g @ W0v.T)[None], HP)
    a4 = _padlane((lb @ W0v.T + b0v)[None], HP)
    a34 = jnp.concatenate([a3, a4], axis=0)
    mw = wmat.mean(1)
    mb = bmat.mean(1)
    vw = wmat.var(1)
    vb = bmat.var(1)
    cwb = (wmat * bmat).mean(1) - mw * mb
    mom = jnp.stack([mw, mb, vw, cwb, vb, sguw[1:NNUM + 1],
                     jnp.zeros_like(mw), jnp.zeros_like(mw)])
    wg = _padlane(sguw[1:NNUM + 1, None] * sgu_g[None], HP)

    w0vp = jnp.pad(W0v, ((0, HP - H), (0, 0)))
    b0vp = _padlane(b0v[None], HP)
    sgugb = jnp.concatenate([_padlane(sgu_g[None], HP),
                             _padlane(sgu_b[None], HP)], axis=0)
    scal = sguw[NNUM + 1:].astype(jnp.float32)
    bias8 = p['tok_bias'][NNUM:]

    numv = _run_numeric(x_num.astype(jnp.float32), a1, a2, a34, wg, mom,
                        _padlane(const[None], HP))

    table = _build_table(p['cat_emb'], bias8, g[None], lb[None], w0vp, b0vp,
                         sgugb, scal)

    fidx = (x_cat.astype(jnp.int32)
            + (jnp.arange(NCAT, dtype=jnp.int32) * CARD)[None]).reshape(-1)
    catc = _gather_fn(table, fidx)

    u0p = _padlane(u0[None], HP)
    w1p = _padlane(p['W1'], HP).T
    sc3 = jnp.concatenate([p['base_b'], p['gb2'], p['ab2'],
                           jnp.zeros((5,), jnp.float32)])

    return _run_heads(numv, catc, u0p, w1p, (p['b1'] + row0)[None], g[None],
                      lb[None], p['base_W'], p['gW1'], p['gb1'][None],
                      p['gW2'], p['rW1'], p['rb1'][None], p['rW2'],
                      p['aW1'], p['ab1'][None], p['aW2'], sc3,
                      p['rb2'][None], p['eb2'], p['eW1'], p['eb1'], p['eW2'])
